# Initial kernel scaffold; baseline (speedup 1.0000x reference)
import jax
import jax.numpy as jnp
from jax.experimental import pallas as pl


def kernel(x, edge_index, W, a):
    raise NotImplementedError("write your pallas kernel here")


# speedup vs baseline: 16.5119x; 16.5119x over previous
import functools

import jax
import jax.numpy as jnp
from jax import lax
from jax.experimental import pallas as pl
from jax.experimental.pallas import tpu as pltpu
from jax.experimental.pallas import tpu_sc as plsc

N_HEADS = 4
OUT_F = 32
HF = N_HEADS * OUT_F
P = 144
NC, NS = 2, 16
NW = NC * NS
C = 80
LANES = 16


def _tc_prep_body(x_ref, wt_ref, aa_ref, wh_ref, al_ref):
    wh = jnp.dot(x_ref[...], wt_ref[...], preferred_element_type=jnp.float32)
    wh_ref[...] = wh
    al_ref[...] = jnp.dot(wh, aa_ref[...], preferred_element_type=jnp.float32)


def _tc_finish_body(p_ref, s_ref, o_ref):
    tot = p_ref[0] + p_ref[1]
    numer = tot[:, :HF]
    den = tot[:, HF:HF + N_HEADS]
    mult = jnp.dot(1.0 / (den + 1e-10), s_ref[...],
                   preferred_element_type=jnp.float32)
    o_ref[...] = numer * mult


def _sc_agg_body(wh_hbm, al_hbm, src_hbm, dst_hbm, out_hbm,
                 asrc_v, adst_v, rows_v, obuf_v, sidx_v, didx_v, numer_s, sem):
    n = wh_hbm.shape[0]
    e_total = src_hbm.shape[0]
    epw = e_total // NW
    nchunk = epw // C
    c = lax.axis_index("c")
    s = lax.axis_index("s")
    w = c * NS + s
    ebase = w * epw

    rpt = n // NS
    row0 = s * rpt

    zeros16 = jnp.zeros((LANES,), jnp.float32)

    def _zero_body(i, _):
        r = i // (P // LANES)
        j = i % (P // LANES)
        obuf_v[r, pl.ds(j * LANES, LANES)] = zeros16
        return 0

    lax.fori_loop(0, C * (P // LANES), _zero_body, 0)

    nfull = rpt // C
    rem = rpt - nfull * C

    def _zcopy(j, _):
        pltpu.sync_copy(obuf_v, numer_s.at[pl.ds(row0 + j * C, C)])
        return 0

    lax.fori_loop(0, nfull, _zcopy, 0)
    pltpu.sync_copy(obuf_v.at[pl.ds(0, rem)],
                    numer_s.at[pl.ds(row0 + nfull * C, rem)])

    plsc.subcore_barrier()

    lanes = lax.iota(jnp.int32, LANES)

    def _chunk_body(t, _):
        base = ebase + t * C
        pltpu.sync_copy(src_hbm.at[pl.ds(base, C)], sidx_v)
        pltpu.sync_copy(dst_hbm.at[pl.ds(base, C)], didx_v)
        pltpu.async_copy(wh_hbm.at[sidx_v], rows_v, sem).wait()
        pltpu.async_copy(al_hbm.at[sidx_v], asrc_v, sem).wait()
        pltpu.async_copy(al_hbm.at[didx_v], adst_v, sem).wait()

        def _group_body(g, _):
            erow = g * LANES + lanes
            exs = []
            for h in range(N_HEADS):
                ch = jnp.full((LANES,), h, jnp.int32)
                ea = plsc.load_gather(asrc_v, [erow, ch])
                eb = plsc.load_gather(adst_v, [erow, ch + N_HEADS])
                ev = ea + eb
                ev = jnp.where(ev > 0, ev, ev * 0.2)
                ex = jnp.exp(ev)
                plsc.store_scatter(
                    obuf_v, [erow, jnp.full((LANES,), HF + h, jnp.int32)], ex)
                exs.append(ex)
            for h in range(N_HEADS):
                for f in range(OUT_F):
                    col = jnp.full((LANES,), h * OUT_F + f, jnp.int32)
                    v = plsc.load_gather(rows_v, [erow, col])
                    plsc.store_scatter(obuf_v, [erow, col], v * exs[h])
            return 0

        lax.fori_loop(0, C // LANES, _group_body, 0)
        pltpu.sync_copy(obuf_v, numer_s.at[didx_v], add=True)
        return 0

    lax.fori_loop(0, nchunk, _chunk_body, 0)

    plsc.subcore_barrier()

    def _ocopy(j, _):
        r0 = row0 + j * C
        pltpu.sync_copy(numer_s.at[pl.ds(r0, C)], obuf_v)
        pltpu.sync_copy(obuf_v, out_hbm.at[c, pl.ds(r0, C)])
        return 0

    lax.fori_loop(0, nfull, _ocopy, 0)
    r0 = row0 + nfull * C
    pltpu.sync_copy(numer_s.at[pl.ds(r0, rem)], obuf_v.at[pl.ds(0, rem)])
    pltpu.sync_copy(obuf_v.at[pl.ds(0, rem)], out_hbm.at[c, pl.ds(r0, rem)])


def kernel(x, edge_index, W, a):
    n, in_f = x.shape
    e = edge_index.shape[1]
    src = edge_index[0]
    dst = edge_index[1]
    wt = W.T

    onehot = jnp.eye(N_HEADS, dtype=x.dtype)
    a_src = a[:, :OUT_F]
    a_dst = a[:, OUT_F:]
    aa = jnp.concatenate(
        [a_src[:, :, None] * onehot[:, None, :],
         a_dst[:, :, None] * onehot[:, None, :]], axis=-1).reshape(HF, 2 * N_HEADS)

    sel = jnp.kron(jnp.eye(N_HEADS, dtype=x.dtype), jnp.ones((1, OUT_F), x.dtype))

    bn = 1000
    wh, al = pl.pallas_call(
        _tc_prep_body,
        grid=(n // bn,),
        in_specs=[
            pl.BlockSpec((bn, in_f), lambda i: (i, 0)),
            pl.BlockSpec((in_f, HF), lambda i: (0, 0)),
            pl.BlockSpec((HF, 2 * N_HEADS), lambda i: (0, 0)),
        ],
        out_specs=[
            pl.BlockSpec((bn, HF), lambda i: (i, 0)),
            pl.BlockSpec((bn, 2 * N_HEADS), lambda i: (i, 0)),
        ],
        out_shape=[
            jax.ShapeDtypeStruct((n, HF), jnp.float32),
            jax.ShapeDtypeStruct((n, 2 * N_HEADS), jnp.float32),
        ],
    )(x, wt, aa)

    mesh = plsc.VectorSubcoreMesh(core_axis_name="c", subcore_axis_name="s",
                                  num_cores=NC, num_subcores=NS)
    sc_agg = pl.kernel(
        _sc_agg_body,
        out_type=jax.ShapeDtypeStruct((NC, n, P), jnp.float32),
        mesh=mesh,
        compiler_params=pltpu.CompilerParams(use_tc_tiling_on_sc=False,
                                              needs_layout_passes=False),
        scratch_types=[
            pltpu.VMEM((C, 2 * N_HEADS), jnp.float32),
            pltpu.VMEM((C, 2 * N_HEADS), jnp.float32),
            pltpu.VMEM((C, HF), jnp.float32),
            pltpu.VMEM((C, P), jnp.float32),
            pltpu.VMEM((C,), jnp.int32),
            pltpu.VMEM((C,), jnp.int32),
            pltpu.VMEM_SHARED((n, P), jnp.float32),
            pltpu.SemaphoreType.DMA,
        ],
    )
    partials = sc_agg(wh, al, src, dst)

    out = pl.pallas_call(
        _tc_finish_body,
        grid=(n // bn,),
        in_specs=[
            pl.BlockSpec((NC, bn, P), lambda i: (0, i, 0)),
            pl.BlockSpec((N_HEADS, HF), lambda i: (0, 0)),
        ],
        out_specs=pl.BlockSpec((bn, HF), lambda i: (i, 0)),
        out_shape=jax.ShapeDtypeStruct((n, HF), jnp.float32),
    )(partials, sel)
    return out
